# Initial kernel scaffold; baseline (speedup 1.0000x reference)
import jax
import jax.numpy as jnp
from jax.experimental import pallas as pl


def kernel(x_h, edge_index_h, edge_attr_h, batch_h, x_t, edge_index_t, edge_attr_t, batch_t, params):
    raise NotImplementedError("write your pallas kernel here")


# speedup vs baseline: 2.7906x; 2.7906x over previous
import functools

import jax
import jax.numpy as jnp
from jax import lax
from jax.experimental import pallas as pl
from jax.experimental.pallas import tpu as pltpu
from jax.experimental.pallas import tpu_sc as plsc

N = 10000
E = 320000
D = 128
B = 64
L = 2
K = N // 2
ATOM_VOCAB = 120
BOND_VOCAB = 8
NUP = L * (L + 1) // 2

NC = 2
NS = 16
NW = NC * NS
EPW = E // NW
CH = 80
NCH = EPW // CH
RPW = 624
TAIL = N - NS * RPW

RB = 1000
G = N // RB

f32 = jnp.float32
i32 = jnp.int32


def _sc_agg_body(table, src_hbm, dst_hbm, zeros_hbm, out_hbm,
                 src_v, dst_v, rows_v, agg_sh, sem):
    cid = lax.axis_index("c")
    sid = lax.axis_index("s")
    wid = sid * NC + cid

    rbase = pl.multiple_of(sid * RPW, 8)
    pltpu.sync_copy(zeros_hbm.at[pl.ds(rbase, RPW)],
                    agg_sh.at[pl.ds(rbase, RPW)])

    @pl.when(sid == NS - 1)
    def _():
        pltpu.sync_copy(zeros_hbm.at[pl.ds(NS * RPW, TAIL)],
                        agg_sh.at[pl.ds(NS * RPW, TAIL)])

    pltpu.sync_copy(src_hbm.at[wid], src_v)
    pltpu.sync_copy(dst_hbm.at[wid], dst_v)
    plsc.subcore_barrier()

    def chunk(k, carry):
        pltpu.async_copy(table.at[src_v.at[k]], rows_v, sem).wait()
        pltpu.sync_copy(rows_v, agg_sh.at[dst_v.at[k]], add=True)
        return carry

    lax.fori_loop(0, NCH, chunk, 0)
    plsc.subcore_barrier()
    obase = pl.multiple_of(cid * N + sid * RPW, 8)
    pltpu.sync_copy(agg_sh.at[pl.ds(rbase, RPW)],
                    out_hbm.at[pl.ds(obase, RPW)])

    @pl.when(sid == NS - 1)
    def _():
        tbase = pl.multiple_of(cid * N + NS * RPW, 8)
        pltpu.sync_copy(agg_sh.at[pl.ds(NS * RPW, TAIL)],
                        out_hbm.at[pl.ds(tbase, TAIL)])


@functools.cache
def _sc_agg_kernel(table_rows: int):
    mesh = plsc.VectorSubcoreMesh(core_axis_name="c", subcore_axis_name="s",
                                  num_cores=NC, num_subcores=NS)
    return pl.kernel(
        _sc_agg_body,
        out_type=jax.ShapeDtypeStruct((NC * N, D), f32),
        mesh=mesh,
        scratch_types=[
            pltpu.VMEM((NCH, CH), i32),
            pltpu.VMEM((NCH, CH), i32),
            pltpu.VMEM((CH, D), f32),
            pltpu.VMEM_SHARED((N, D), f32),
            pltpu.SemaphoreType.DMA,
        ],
    )


def _adj_matvec_parts(table, src3, dst3, zeros):
    return _sc_agg_kernel(table.shape[0])(table, src3, dst3, zeros)


def _row_spec(shape=(RB, D)):
    return pl.BlockSpec(shape, lambda j: (j, 0))


def _full_spec(shape):
    return pl.BlockSpec(shape, lambda j: tuple(0 for _ in shape))


def _tc_embed(ids_col, emb):
    V = emb.shape[0]

    def body(ids_ref, emb_ref, out_ref):
        ids = ids_ref[...]
        oh = (ids == lax.broadcasted_iota(i32, (RB, V), 1)).astype(f32)
        out_ref[...] = jnp.dot(oh, emb_ref[...],
                               preferred_element_type=f32)

    return pl.pallas_call(
        body,
        grid=(G,),
        in_specs=[_row_spec((RB, 1)), _full_spec((V, D))],
        out_specs=_row_spec(),
        out_shape=jax.ShapeDtypeStruct((N, D), f32),
    )(ids_col, emb)


def _tc_mp_post(y, parts, hea, W, b, jump=None):
    with_jump = jump is not None

    def body(*refs):
        if with_jump:
            (y_ref, p0_ref, p1_ref, h0_ref, h1_ref, w_ref, b_ref, j_ref,
             out_ref, jo_ref) = refs
        else:
            (y_ref, p0_ref, p1_ref, h0_ref, h1_ref, w_ref, b_ref,
             out_ref) = refs
        acc = (y_ref[...] + p0_ref[...] + p1_ref[...]
               + h0_ref[...] + h1_ref[...])
        h = jnp.maximum(
            jnp.dot(acc, w_ref[...], preferred_element_type=f32)
            + b_ref[...], 0.0)
        out_ref[...] = h
        if with_jump:
            jo_ref[...] = j_ref[...] + h

    part0 = pl.BlockSpec((RB, D), lambda j: (j, 0))
    part1 = pl.BlockSpec((RB, D), lambda j: (j + G, 0))
    in_specs = [_row_spec(), part0, part1, part0, part1,
                _full_spec((D, D)), _full_spec((1, D))]
    operands = [y, parts, parts, hea, hea, W, b.reshape(1, D)]
    if with_jump:
        in_specs.append(_row_spec())
        operands.append(jump)
        out_specs = (_row_spec(), _row_spec())
        out_shape = (jax.ShapeDtypeStruct((N, D), f32),
                     jax.ShapeDtypeStruct((N, D), f32))
    else:
        out_specs = _row_spec()
        out_shape = jax.ShapeDtypeStruct((N, D), f32)
    return pl.pallas_call(
        body, grid=(G,), in_specs=in_specs, out_specs=out_specs,
        out_shape=out_shape,
    )(*operands)


def _tc_mean_sums(x, bcol):
    def body(x_ref, b_ref, sum_ref, cnt_ref):
        j = pl.program_id(0)

        @pl.when(j == 0)
        def _():
            sum_ref[...] = jnp.zeros_like(sum_ref)
            cnt_ref[...] = jnp.zeros_like(cnt_ref)

        oh = (b_ref[...] == lax.broadcasted_iota(i32, (RB, B), 1)).astype(f32)
        sum_ref[...] += lax.dot_general(
            oh, x_ref[...], (((0,), (0,)), ((), ())),
            preferred_element_type=f32)
        cnt_ref[...] += lax.dot_general(
            oh, jnp.ones((RB, 1), f32), (((0,), (0,)), ((), ())),
            preferred_element_type=f32)

    return pl.pallas_call(
        body,
        grid=(G,),
        in_specs=[_row_spec(), _row_spec((RB, 1))],
        out_specs=(_full_spec((B, D)), _full_spec((B, 1))),
        out_shape=(jax.ShapeDtypeStruct((B, D), f32),
                   jax.ShapeDtypeStruct((B, 1), f32)),
    )(x, bcol)


def _tc_pool_score(x, bcol, qsum, qcnt, gate):
    def body(x_ref, b_ref, qs_ref, qc_ref, g_ref, s_ref):
        q = qs_ref[...] / jnp.maximum(qc_ref[...], 1.0)
        oh = (b_ref[...] == lax.broadcasted_iota(i32, (RB, B), 1)).astype(f32)
        qb = jnp.dot(oh, q, preferred_element_type=f32)
        xx = x_ref[...]
        dotq = jnp.sum(xx * qb, axis=1, keepdims=True)
        gv = lax.dot_general(xx, g_ref[...], (((1,), (1,)), ((), ())),
                             preferred_element_type=f32)
        s_ref[...] = dotq * (1.0 / (D ** 0.5)) + gv

    return pl.pallas_call(
        body,
        grid=(G,),
        in_specs=[_row_spec(), _row_spec((RB, 1)), _full_spec((B, D)),
                  _full_spec((B, 1)), _full_spec((1, D))],
        out_specs=_row_spec((RB, 1)),
        out_shape=jax.ShapeDtypeStruct((N, 1), f32),
    )(x, bcol, qsum, qcnt, gate.reshape(1, D))


_SPAD_R = 80


def _tc_pool_apply(spad, x, scol):
    def body(sp_ref, x_ref, s_ref, xp_ref, m_ref, kth_ref):
        j = pl.program_id(0)

        @pl.when(j == 0)
        def _():
            bits = lax.bitcast_convert_type(sp_ref[...], i32)
            key = jnp.where(bits >= 0, bits, bits ^ jnp.int32(0x7FFFFFFF))
            ku = (lax.bitcast_convert_type(key, jnp.uint32)
                  + jnp.uint32(0x80000000))

            def bit_step(it, prefix):
                b = (31 - it).astype(jnp.uint32)
                cand = prefix | (jnp.uint32(1) << b)
                cnt = jnp.sum(jnp.where(ku >= cand, 1.0, 0.0))
                return jnp.where(cnt >= K, cand, prefix)

            prefix = lax.fori_loop(0, 32, bit_step, jnp.uint32(0))
            ki = lax.bitcast_convert_type(prefix + jnp.uint32(0x80000000),
                                          i32)
            ubits = jnp.where(ki >= 0, ki, ki ^ jnp.int32(0x7FFFFFFF))
            kth_ref[0] = lax.bitcast_convert_type(ubits, f32)

        kth = kth_ref[0]
        s = s_ref[...]
        mask = (s >= kth).astype(f32)
        gatev = mask / (1.0 + jnp.exp(-s))
        xp_ref[...] = x_ref[...] * gatev
        m_ref[...] = mask

    return pl.pallas_call(
        body,
        grid=(G,),
        in_specs=[_full_spec((_SPAD_R, D)), _row_spec(), _row_spec((RB, 1))],
        out_specs=(_row_spec(), _row_spec((RB, 1))),
        out_shape=(jax.ShapeDtypeStruct((N, D), f32),
                   jax.ShapeDtypeStruct((N, 1), f32)),
        scratch_shapes=[pltpu.SMEM((1,), f32)],
    )(spad, x, scol)


def _tc_unpool(hp, buf, mask, jump, wrecip):
    def body(hp_ref, buf_ref, m_ref, j_ref, out_ref):
        m = m_ref[...] > 0.0
        out_ref[...] = (jnp.where(m, hp_ref[...], buf_ref[...])
                        + j_ref[...] * wrecip)

    return pl.pallas_call(
        body,
        grid=(G,),
        in_specs=[_row_spec(), _row_spec(), _row_spec((RB, 1)), _row_spec()],
        out_specs=_row_spec(),
        out_shape=jax.ShapeDtypeStruct((N, D), f32),
    )(hp, buf, mask, jump)


def _tc_jk(e0, e1, W1, b1, W2, b2):
    def body(e0_ref, e1_ref, w1_ref, b1_ref, w2_ref, b2_ref, out_ref):
        w1 = w1_ref[...]
        b1v = b1_ref[...]
        w2 = w2_ref[...]
        b2v = b2_ref[...]
        x0 = e0_ref[...]
        x1 = e1_ref[...]
        a0 = jnp.dot(jnp.maximum(jnp.dot(x0, w1,
                                         preferred_element_type=f32) + b1v,
                                 0.0), w2, preferred_element_type=f32) + b2v
        a1 = jnp.dot(jnp.maximum(jnp.dot(x1, w1,
                                         preferred_element_type=f32) + b1v,
                                 0.0), w2, preferred_element_type=f32) + b2v
        m = jnp.maximum(a0, a1)
        z0 = jnp.exp(a0 - m)
        z1 = jnp.exp(a1 - m)
        out_ref[...] = (z0 * x0 + z1 * x1) / (z0 + z1)

    return pl.pallas_call(
        body,
        grid=(G,),
        in_specs=[_row_spec(), _row_spec(), _full_spec((D, D)),
                  _full_spec((1, D)), _full_spec((D, 1)), _full_spec((1, 1))],
        out_specs=_row_spec(),
        out_shape=jax.ShapeDtypeStruct((N, D), f32),
    )(e0, e1, W1, b1.reshape(1, D), W2, b2.reshape(1, 1))


def _tc_final(hs, hc, ts, tc, W1, b1, W2, b2, pW, pb):
    def body(hs_ref, hc_ref, ts_ref, tc_ref, w1_ref, b1_ref, w2_ref, b2_ref,
             pw_ref, pb_ref, out_ref):
        def dec(s_ref, c_ref):
            g = s_ref[...] / jnp.maximum(c_ref[...], 1.0)
            h1 = jnp.maximum(
                jnp.dot(g, w1_ref[...], preferred_element_type=f32)
                + b1_ref[...], 0.0)
            return (jnp.dot(h1, w2_ref[...], preferred_element_type=f32)
                    + b2_ref[...])

        gh = dec(hs_ref, hc_ref)
        gt = dec(ts_ref, tc_ref)
        out_ref[...] = (jnp.dot(gh * gt, pw_ref[...],
                                preferred_element_type=f32) + pb_ref[...])

    return pl.pallas_call(
        body,
        in_specs=[pl.BlockSpec((B, D), lambda: (0, 0)),
                  pl.BlockSpec((B, 1), lambda: (0, 0)),
                  pl.BlockSpec((B, D), lambda: (0, 0)),
                  pl.BlockSpec((B, 1), lambda: (0, 0)),
                  pl.BlockSpec((D, 2 * D), lambda: (0, 0)),
                  pl.BlockSpec((1, 2 * D), lambda: (0, 0)),
                  pl.BlockSpec((2 * D, D), lambda: (0, 0)),
                  pl.BlockSpec((1, D), lambda: (0, 0)),
                  pl.BlockSpec((D, 1), lambda: (0, 0)),
                  pl.BlockSpec((1, 1), lambda: (0, 0))],
        out_specs=pl.BlockSpec((B, 1), lambda: (0, 0)),
        out_shape=jax.ShapeDtypeStruct((B, 1), f32),
    )(hs, hc, ts, tc, W1, b1.reshape(1, 2 * D), W2, b2.reshape(1, D),
      pW, pb.reshape(1, 1))


def _pad_scores(s):
    flat = jnp.concatenate(
        [s.reshape(-1), jnp.full((_SPAD_R * D - N,), -jnp.inf, f32)])
    return flat.reshape(_SPAD_R, D)


def kernel(x_h, edge_index_h, edge_attr_h, batch_h,
           x_t, edge_index_t, edge_attr_t, batch_t, params):
    p = params
    zeros = jnp.zeros((N, D), f32)

    def prep(xi, ei, eai, bi):
        ids = xi.astype(i32).reshape(N, 1)
        src = ei[0].astype(i32).reshape(NW, NCH, CH)
        dst = ei[1].astype(i32).reshape(NW, NCH, CH)
        ea3 = eai.astype(i32).reshape(NW, NCH, CH)
        bcol = bi.astype(i32).reshape(N, 1)
        return ids, src, dst, ea3, bcol

    ids_h, src_h, dst_h, ea_h3, bcol_h = prep(x_h, edge_index_h,
                                              edge_attr_h, batch_h)
    ids_t, src_t, dst_t, ea_t3, bcol_t = prep(x_t, edge_index_t,
                                              edge_attr_t, batch_t)

    hea_h = _adj_matvec_parts(p["bond_emb"], ea_h3, dst_h, zeros)
    hea_t = _adj_matvec_parts(p["bond_emb"], ea_t3, dst_t, zeros)

    def mp(y, src, dst, hea, W, b, jump=None):
        parts = _adj_matvec_parts(y, src, dst, zeros)
        return _tc_mp_post(y, parts, hea, W, b, jump=jump)

    hx = _tc_embed(ids_h, p["atom_emb"])
    tx = _tc_embed(ids_t, p["atom_emb"])
    hx = mp(hx, src_h, dst_h, hea_h, p["W_init"], p["b_init"])
    tx = mp(tx, src_t, dst_t, hea_t, p["W_init"], p["b_init"])

    x_jump_h, x_jump_t = [], []
    emx_h, emx_t = [], []
    buf_h, buf_t, masks_h, masks_t = [], [], [], []
    for i in range(L):
        x_jump_h.append(hx)
        x_jump_t.append(tx)
        qs_h, qc_h = _tc_mean_sums(hx, bcol_h)
        qs_t, qc_t = _tc_mean_sums(tx, bcol_t)
        buf_h.append(hx)
        buf_t.append(tx)
        s_h = _tc_pool_score(hx, bcol_h, qs_t, qc_t, p["gate"])
        s_t = _tc_pool_score(tx, bcol_t, qs_h, qc_h, p["gate"])
        hx, m_h = _tc_pool_apply(_pad_scores(s_h), hx, s_h)
        tx, m_t = _tc_pool_apply(_pad_scores(s_t), tx, s_t)
        masks_h.append(m_h)
        masks_t.append(m_t)
        hx = mp(hx, src_h, dst_h, hea_h, p["W_down"][i], p["b_down"][i])
        tx = mp(tx, src_t, dst_t, hea_t, p["W_down"][i], p["b_down"][i])
        hp, tp = hx, tx
        w = 1.0
        for j in range(i, -1, -1):
            hp_in = _tc_unpool(hp, buf_h[j], masks_h[j], x_jump_h[j], 1.0 / w)
            tp_in = _tc_unpool(tp, buf_t[j], masks_t[j], x_jump_t[j], 1.0 / w)
            idx = i * (i + 1) // 2 + j
            hp, x_jump_h[j] = mp(hp_in, src_h, dst_h, hea_h,
                                 p["W_up"][idx], p["b_up"][idx],
                                 jump=x_jump_h[j])
            tp, x_jump_t[j] = mp(tp_in, src_t, dst_t, hea_t,
                                 p["W_up"][idx], p["b_up"][idx],
                                 jump=x_jump_t[j])
            w += 1.0
        emx_h.append(hp)
        emx_t.append(tp)

    xh = _tc_jk(emx_h[0], emx_h[1], p["jk_W1"], p["jk_b1"],
                p["jk_W2"], p["jk_b2"])
    xt = _tc_jk(emx_t[0], emx_t[1], p["jk_W1"], p["jk_b1"],
                p["jk_W2"], p["jk_b2"])

    hs, hc = _tc_mean_sums(xh, bcol_h)
    ts, tc = _tc_mean_sums(xt, bcol_t)

    return _tc_final(hs, hc, ts, tc, p["dec_W1"], p["dec_b1"],
                     p["dec_W2"], p["dec_b2"], p["pred_W"], p["pred_b"])
